# baseline (speedup 1.0000x reference)
import functools

import jax
import jax.numpy as jnp
from jax import lax
from jax.experimental import pallas as pl
from jax.experimental.pallas import tpu as pltpu
from jax.experimental.pallas import tpu_sc as plsc

_NUM_CLASSES = 1000
_HIDDEN = 1152
_BATCH = 16384
_ROWS = _NUM_CLASSES + 1

_NC = 2
_NS = 16
_NW = _NC * _NS
_BPW = _BATCH // _NW
_GRP = 16
_NGRP = _BPW // _GRP
_NBUF = 3

_STG = 64
_STG_LAST = _ROWS - 15 * _STG

_mesh = plsc.VectorSubcoreMesh(core_axis_name="c", subcore_axis_name="s")


@functools.partial(
    pl.kernel,
    mesh=_mesh,
    out_type=jax.ShapeDtypeStruct((_BATCH, _HIDDEN), jnp.float32),
    scratch_types=[
        pltpu.VMEM_SHARED((_ROWS, _HIDDEN), jnp.float32),
        pltpu.VMEM((_NBUF, _GRP, _HIDDEN), jnp.float32),
        pltpu.VMEM((_BPW,), jnp.int32),
        pltpu.VMEM((_BPW,), jnp.int32),
        *[pltpu.SemaphoreType.DMA for _ in range(_NBUF)],
        *[pltpu.SemaphoreType.DMA for _ in range(_NBUF)],
    ],
    compiler_params=pltpu.CompilerParams(
        use_tc_tiling_on_sc=False, needs_layout_passes=False),
)
def _embed(labels_hbm, force_hbm, table_hbm, out_hbm,
           table_sp, rowbuf, idx_v, frc_v, *sems):
    rsems = sems[:_NBUF]
    osems = sems[_NBUF:]
    cid = lax.axis_index("c")
    sid = lax.axis_index("s")
    wid = sid * _NC + cid
    base = pl.multiple_of(wid * _BPW, _BPW)

    @pl.when(sid < 15)
    def _():
        off = pl.multiple_of(sid * _STG, _STG)
        pltpu.sync_copy(table_hbm.at[pl.ds(off, _STG)],
                        table_sp.at[pl.ds(off, _STG)])

    @pl.when(sid == 15)
    def _():
        pltpu.sync_copy(table_hbm.at[pl.ds(15 * _STG, _STG_LAST)],
                        table_sp.at[pl.ds(15 * _STG, _STG_LAST)])

    pltpu.sync_copy(labels_hbm.at[pl.ds(base, _BPW)], idx_v)
    pltpu.sync_copy(force_hbm.at[pl.ds(base, _BPW)], frc_v)

    for i in range(_BPW // 16):
        sl = pl.ds(i * 16, 16)
        idx_v[sl] = jnp.where(frc_v[sl] == 1, _NUM_CLASSES, idx_v[sl])

    plsc.subcore_barrier()

    def issue_rows(g, b):
        idx16 = idx_v[pl.ds(g * _GRP, _GRP)]
        for r in range(_GRP):
            pltpu.async_copy(table_sp.at[idx16[r]], rowbuf.at[b, r], rsems[b])

    def wait_rows(b):
        for r in range(_GRP):
            pltpu.make_async_copy(
                table_hbm.at[0], rowbuf.at[b, r], rsems[b]).wait()

    def drain_out(b):
        pltpu.make_async_copy(
            out_hbm.at[pl.ds(0, _GRP)], rowbuf.at[b], osems[b]).wait()

    issue_rows(0, 0)
    for g in range(_NGRP):
        if g >= 2:
            drain_out((g - 2) % _NBUF)
        if g + 1 < _NGRP:
            issue_rows(g + 1, (g + 1) % _NBUF)
        b = g % _NBUF
        wait_rows(b)
        pltpu.async_copy(rowbuf.at[b],
                         out_hbm.at[pl.ds(base + g * _GRP, _GRP)], osems[b])

    for g in range(_NGRP - 2, _NGRP):
        drain_out(g % _NBUF)


def kernel(labels, train, force_drop_ids, embedding_table):
    del train
    return _embed(labels.astype(jnp.int32),
                  force_drop_ids.astype(jnp.int32),
                  embedding_table)
